# baseline (speedup 1.0000x reference)
import functools

import jax
import jax.numpy as jnp
from jax import lax
from jax.experimental import pallas as pl
from jax.experimental.pallas import tpu as pltpu
from jax.experimental.pallas import tpu_sc as plsc

BATCH = 16384
EMBED_DIM = 64
NROWS = 1000000

_info = plsc.get_sparse_core_info()
_NC, _NS, _L = _info.num_cores, _info.num_subcores, _info.num_lanes
_NW = _NC * _NS
_BPW = BATCH // _NW
_CHUNK = 256
_NCHUNK = _BPW // _CHUNK
_GPC = _CHUNK // _L


def _gmf_body(user_id_hbm, item_id_hbm, ut_hbm, it_hbm,
              w_hbm, out_hbm, ids_u, ids_i, idx_u, idx_i,
              g_u, g_i, w_v, out_v, sem):
    wid = lax.axis_index("s") * _NC + lax.axis_index("c")
    base = wid * _BPW

    pltpu.sync_copy(user_id_hbm.at[pl.ds(base, _BPW)], ids_u)
    pltpu.sync_copy(item_id_hbm.at[pl.ds(base, _BPW)], ids_i)
    pltpu.sync_copy(w_hbm, w_v)

    lanes = lax.iota(jnp.int32, _L)

    def chunk_body(k, _):
        def ridx_body(g, _):
            u16 = ids_u[pl.ds(k * _CHUNK + g * _L, _L)]
            i16 = ids_i[pl.ds(k * _CHUNK + g * _L, _L)]
            idx_u[pl.ds(g * _L, _L)] = lax.shift_right_logical(u16, 1)
            idx_i[pl.ds(g * _L, _L)] = lax.shift_right_logical(i16, 1)
            return 0
        lax.fori_loop(0, _GPC, ridx_body, 0)

        cu = pltpu.async_copy(ut_hbm.at[idx_u], g_u, sem)
        ci = pltpu.async_copy(it_hbm.at[idx_i], g_i, sem)
        cu.wait()
        ci.wait()

        def group_body(g, _):
            u16 = ids_u[pl.ds(k * _CHUNK + g * _L, _L)]
            i16 = ids_i[pl.ds(k * _CHUNK + g * _L, _L)]
            uoff = jnp.bitwise_and(u16, 1) * EMBED_DIM
            ioff = jnp.bitwise_and(i16, 1) * EMBED_DIM
            jj = g * _L + lanes

            def d_body(d, acc):
                dd = jnp.full((_L,), d, jnp.int32)
                ug = plsc.load_gather(g_u, [jj, uoff + dd])
                ig = plsc.load_gather(g_i, [jj, ioff + dd])
                wg = plsc.load_gather(w_v, [dd])
                return acc + ug * ig * wg

            acc = lax.fori_loop(0, EMBED_DIM, d_body,
                                jnp.zeros((_L,), jnp.float32))
            out_v[pl.ds(k * _CHUNK + g * _L, _L)] = acc
            return 0
        lax.fori_loop(0, _GPC, group_body, 0)
        return 0

    lax.fori_loop(0, _NCHUNK, chunk_body, 0)

    pltpu.sync_copy(out_v, out_hbm.at[pl.ds(base, _BPW)])


@jax.jit
def _gmf(user_id, item_id, user_table, item_table, predict_w):
    mesh = plsc.VectorSubcoreMesh(core_axis_name="c", subcore_axis_name="s")
    ut2 = user_table.reshape(NROWS // 2, 2 * EMBED_DIM)
    it2 = item_table.reshape(NROWS // 2, 2 * EMBED_DIM)
    return pl.kernel(
        _gmf_body,
        mesh=mesh,
        compiler_params=pltpu.CompilerParams(needs_layout_passes=False),
        out_type=jax.ShapeDtypeStruct((BATCH,), jnp.float32),
        scratch_types=[
            pltpu.VMEM((_BPW,), jnp.int32),
            pltpu.VMEM((_BPW,), jnp.int32),
            pltpu.VMEM((_CHUNK,), jnp.int32),
            pltpu.VMEM((_CHUNK,), jnp.int32),
            pltpu.VMEM((_CHUNK, 2 * EMBED_DIM), jnp.float32),
            pltpu.VMEM((_CHUNK, 2 * EMBED_DIM), jnp.float32),
            pltpu.VMEM((EMBED_DIM,), jnp.float32),
            pltpu.VMEM((_BPW,), jnp.float32),
            pltpu.SemaphoreType.DMA,
        ],
    )(user_id, item_id, ut2, it2, predict_w)


def kernel(user_id, item_id, user_table, item_table, predict_w):
    return _gmf(user_id.astype(jnp.int32), item_id.astype(jnp.int32),
                user_table, item_table, predict_w)


# speedup vs baseline: 1.0270x; 1.0270x over previous
import functools

import jax
import jax.numpy as jnp
from jax import lax
from jax.experimental import pallas as pl
from jax.experimental.pallas import tpu as pltpu
from jax.experimental.pallas import tpu_sc as plsc

BATCH = 16384
EMBED_DIM = 64

_info = plsc.get_sparse_core_info()
_NC, _NS, _L = _info.num_cores, _info.num_subcores, _info.num_lanes
_NW = _NC * _NS
_BPW = BATCH // _NW
_GROUPS = _BPW // _L


def _gather_u_body(user_id_hbm, user_table_hbm, gu_hbm,
                   idx_u, u_rows, sem):
    wid = lax.axis_index("s") * _NC + lax.axis_index("c")
    base = wid * _BPW
    pltpu.sync_copy(user_id_hbm.at[pl.ds(base, _BPW)], idx_u)
    pltpu.async_copy(user_table_hbm.at[idx_u], u_rows, sem).wait()
    pltpu.sync_copy(u_rows, gu_hbm.at[pl.ds(base, _BPW), :])


def _mf_body(item_id_hbm, item_table_hbm, gu_hbm, w_hbm, out_hbm,
             idx_i, u_rows, i_rows, w_v, out_v, sem):
    wid = lax.axis_index("s") * _NC + lax.axis_index("c")
    base = wid * _BPW

    pltpu.sync_copy(item_id_hbm.at[pl.ds(base, _BPW)], idx_i)
    pltpu.sync_copy(w_hbm, w_v)

    ci = pltpu.async_copy(item_table_hbm.at[idx_i], i_rows, sem)
    cu = pltpu.async_copy(gu_hbm.at[pl.ds(base, _BPW), :], u_rows, sem)
    ci.wait()
    cu.wait()

    wc = [w_v[pl.ds(c * _L, _L)] for c in range(EMBED_DIM // _L)]
    lanes = lax.iota(jnp.int32, _L)

    def group_body(g, _):
        vec = jnp.zeros((_L,), jnp.float32)
        for j in range(_L):
            r = g * _L + j
            acc = jnp.zeros((_L,), jnp.float32)
            for c in range(EMBED_DIM // _L):
                uc = u_rows[r, pl.ds(c * _L, _L)]
                ic = i_rows[r, pl.ds(c * _L, _L)]
                acc = acc + uc * ic * wc[c]
            vec = jnp.where(lanes == j, jnp.sum(acc), vec)
        out_v[pl.ds(g * _L, _L)] = vec
        return 0

    lax.fori_loop(0, _GROUPS, group_body, 0)

    pltpu.sync_copy(out_v, out_hbm.at[pl.ds(base, _BPW)])


@jax.jit
def _gmf(user_id, item_id, user_table, item_table, predict_w):
    mesh = plsc.VectorSubcoreMesh(core_axis_name="c", subcore_axis_name="s")
    cp = pltpu.CompilerParams(needs_layout_passes=False,
                              use_tc_tiling_on_sc=False)
    gu = pl.kernel(
        _gather_u_body,
        mesh=mesh,
        compiler_params=cp,
        out_type=jax.ShapeDtypeStruct((BATCH, EMBED_DIM), jnp.float32),
        scratch_types=[
            pltpu.VMEM((_BPW,), jnp.int32),
            pltpu.VMEM((_BPW, EMBED_DIM), jnp.float32),
            pltpu.SemaphoreType.DMA,
        ],
    )(user_id, user_table)

    return pl.kernel(
        _mf_body,
        mesh=mesh,
        compiler_params=cp,
        out_type=jax.ShapeDtypeStruct((BATCH,), jnp.float32),
        scratch_types=[
            pltpu.VMEM((_BPW,), jnp.int32),
            pltpu.VMEM((_BPW, EMBED_DIM), jnp.float32),
            pltpu.VMEM((_BPW, EMBED_DIM), jnp.float32),
            pltpu.VMEM((EMBED_DIM,), jnp.float32),
            pltpu.VMEM((_BPW,), jnp.float32),
            pltpu.SemaphoreType.DMA,
        ],
    )(item_id, item_table, gu, predict_w)


def kernel(user_id, item_id, user_table, item_table, predict_w):
    return _gmf(user_id.astype(jnp.int32), item_id.astype(jnp.int32),
                user_table, item_table, predict_w)


# speedup vs baseline: 1.0297x; 1.0027x over previous
import functools

import jax
import jax.numpy as jnp
from jax import lax
from jax.experimental import pallas as pl
from jax.experimental.pallas import tpu as pltpu
from jax.experimental.pallas import tpu_sc as plsc

BATCH = 16384
EMBED_DIM = 64

_info = plsc.get_sparse_core_info()
_NC, _NS, _L = _info.num_cores, _info.num_subcores, _info.num_lanes
_NW = _NC * _NS
_BPW = BATCH // _NW
_GROUPS = _BPW // _L


def _gmf_body(user_id_hbm, item_id_hbm, user_table_hbm, item_table_hbm,
              w_hbm, out_hbm, idx_u, idx_i, u_rows, i_rows, w_v, out_v, sem):
    wid = lax.axis_index("s") * _NC + lax.axis_index("c")
    base = wid * _BPW

    pltpu.sync_copy(user_id_hbm.at[pl.ds(base, _BPW)], idx_u)
    pltpu.sync_copy(item_id_hbm.at[pl.ds(base, _BPW)], idx_i)
    pltpu.sync_copy(w_hbm, w_v)

    cu = pltpu.async_copy(user_table_hbm.at[idx_u], u_rows, sem)
    ci = pltpu.async_copy(item_table_hbm.at[idx_i], i_rows, sem)
    cu.wait()
    ci.wait()

    wc = [w_v[pl.ds(c * _L, _L)] for c in range(EMBED_DIM // _L)]
    lanes = lax.iota(jnp.int32, _L)

    def group_body(g, _):
        vec = jnp.zeros((_L,), jnp.float32)
        for j in range(_L):
            r = g * _L + j
            acc = jnp.zeros((_L,), jnp.float32)
            for c in range(EMBED_DIM // _L):
                uc = u_rows[r, pl.ds(c * _L, _L)]
                ic = i_rows[r, pl.ds(c * _L, _L)]
                acc = acc + uc * ic * wc[c]
            vec = jnp.where(lanes == j, jnp.sum(acc), vec)
        out_v[pl.ds(g * _L, _L)] = vec
        return 0

    lax.fori_loop(0, _GROUPS, group_body, 0)

    pltpu.sync_copy(out_v, out_hbm.at[pl.ds(base, _BPW)])


@jax.jit
def _gmf(user_id, item_id, user_table, item_table, predict_w):
    mesh = plsc.VectorSubcoreMesh(core_axis_name="c", subcore_axis_name="s")
    return pl.kernel(
        _gmf_body,
        mesh=mesh,
        compiler_params=pltpu.CompilerParams(needs_layout_passes=False,
                                             use_tc_tiling_on_sc=False),
        out_type=jax.ShapeDtypeStruct((BATCH,), jnp.float32),
        scratch_types=[
            pltpu.VMEM((_BPW,), jnp.int32),
            pltpu.VMEM((_BPW,), jnp.int32),
            pltpu.VMEM((_BPW, EMBED_DIM), jnp.float32),
            pltpu.VMEM((_BPW, EMBED_DIM), jnp.float32),
            pltpu.VMEM((EMBED_DIM,), jnp.float32),
            pltpu.VMEM((_BPW,), jnp.float32),
            pltpu.SemaphoreType.DMA,
        ],
    )(user_id, item_id, user_table, item_table, predict_w)


def kernel(user_id, item_id, user_table, item_table, predict_w):
    return _gmf(user_id.astype(jnp.int32), item_id.astype(jnp.int32),
                user_table, item_table, predict_w)
